# Initial kernel scaffold; baseline (speedup 1.0000x reference)
import jax
import jax.numpy as jnp
from jax.experimental import pallas as pl


def kernel(X, Wv, bv, We, be, bn_w, bn_b):
    raise NotImplementedError("write your pallas kernel here")


# speedup vs baseline: 54.5679x; 54.5679x over previous
import functools

import jax
import jax.numpy as jnp
import numpy as np
from jax.experimental import pallas as pl
from jax.experimental.pallas import tpu as pltpu

IN_CH = 64
NODE = 32
K_NEIGS = 8
KS = 5
STRIDE = 2
N = NODE * NODE
B = 4
LAYER_NUM = 8
STEP = 4
ALPHA_V = 0.05
ALPHA_E = 0.9
EPS = 1e-5
E_LOC = 196
E_LOC_PAD = 256
F32_INF_BITS = 0x7F800000


def _local_incidence():
    idx = np.arange(NODE * NODE).reshape(NODE, NODE)
    pats = []
    for i in range(0, NODE - KS + 1, STRIDE):
        for j in range(0, NODE - KS + 1, STRIDE):
            pats.append(idx[i:i + KS, j:j + KS].reshape(-1))
    inp_unf = np.stack(pats)
    e = inp_unf.shape[0]
    H = np.zeros((NODE * NODE, e), dtype=np.float32)
    for k in range(e):
        H[inp_unf[k], k] = 1.0
    return H


def _select_mask(u_ref, kvec_row):
    iota_i = jax.lax.broadcasted_iota(jnp.int32, (N, N), 0)

    def coarse(_, carry):
        lo, hi = carry
        mid = lo + (hi - lo) // 2
        cnt = jnp.sum(jnp.where(u_ref[...] <= mid, 1.0, 0.0), axis=0,
                      keepdims=True)
        ge = cnt >= kvec_row
        return jnp.where(ge, lo, mid + 1), jnp.where(ge, mid, hi)

    lo0 = jnp.zeros((1, N), jnp.int32)
    hi0 = jnp.full((1, N), F32_INF_BITS, jnp.int32)
    _, tstar = jax.lax.fori_loop(0, 31, coarse, (lo0, hi0))

    def fine(_, carry):
        lo, hi = carry
        mid = lo + (hi - lo) // 2
        u = u_ref[...]
        sel = (u < tstar) | ((u == tstar) & (iota_i < mid))
        cnt = jnp.sum(jnp.where(sel, 1.0, 0.0), axis=0, keepdims=True)
        ge = cnt >= kvec_row
        return jnp.where(ge, lo, mid + 1), jnp.where(ge, mid, hi)

    lo0 = jnp.zeros((1, N), jnp.int32)
    hi0 = jnp.full((1, N), N, jnp.int32)
    _, istar = jax.lax.fori_loop(0, 10, fine, (lo0, hi0))

    u = u_ref[...]
    sel = (u < tstar) | ((u == tstar) & (iota_i < istar))
    return jnp.where(sel, 1.0, 0.0)


def _dotT(a, b):
    return jax.lax.dot_general(a, b, (((0,), (0,)), ((), ())),
                               preferred_element_type=jnp.float32)


def _dot(a, b):
    return jax.lax.dot_general(a, b, (((1,), (0,)), ((), ())),
                               preferred_element_type=jnp.float32)


def _bn_feat1(x):
    m = jnp.mean(x, axis=(0, 2), keepdims=True)
    v = jnp.mean((x - m) ** 2, axis=(0, 2), keepdims=True)
    return (x - m) / jnp.sqrt(v + EPS)


def _hgnn_kernel(x_in, wv, bv_row, we, be_row, bnw_row, bnb_row,
                 lh, lht, ldv_col, eye,
                 out_ref,
                 x4, u_ref, mem4, eknn4, eloc4, dvr4):
    diag = (jax.lax.broadcasted_iota(jnp.int32, (N, N), 0) ==
            jax.lax.broadcasted_iota(jnp.int32, (N, N), 1))
    diag_f = jnp.where(diag, 1.0, 0.0)

    x4[...] = x_in[...]

    for layer in range(LAYER_NUM):
        for b in range(B):
            xb = x4[b]
            sq_col = jnp.sum(xb * xb, axis=1, keepdims=True)
            sq_row = _dotT(sq_col, eye[...])
            g = jax.lax.dot_general(xb, xb, (((1,), (1,)), ((), ())),
                                    preferred_element_type=jnp.float32)
            dt = (sq_row + (-2.0) * g) + sq_col
            u = jax.lax.bitcast_convert_type(jnp.maximum(dt, 0.0), jnp.int32)
            u_ref[...] = jnp.maximum(u, 0)

            nine = jnp.full((1, N), 9.0, jnp.float32)
            mask9 = _select_mask(u_ref, nine)
            dv_col = jnp.sum(mask9, axis=1, keepdims=True)
            kvec = _dotT(dv_col, eye[...])
            member = jnp.maximum(_select_mask(u_ref, kvec), diag_f)
            mem4[b] = member

            de_row = jnp.sum(member, axis=0, keepdims=True)
            de_col = jax.lax.dot_general(
                eye[...], de_row, (((1,), (1,)), ((), ())),
                preferred_element_type=jnp.float32)
            eknn4[b] = _dotT(member, xb) * (1.0 / de_col)
            eloc4[b] = _dot(lht[...], xb) * (1.0 / 25.0)
            dv_full = jnp.sum(member, axis=1, keepdims=True) + ldv_col[...]
            dvr4[b] = 1.0 / dv_full

        if layer % STEP == 0:
            xa = x4[...]
            xa = xa + jax.nn.relu(
                jax.lax.dot_general(xa, wv[...], (((2,), (1,)), ((), ())),
                                    preferred_element_type=jnp.float32)
                + bv_row[...][None])
            x4[...] = _bn_feat1(xa)
            for eref in (eknn4, eloc4):
                ea = eref[...]
                ea = ea + jax.nn.relu(
                    jax.lax.dot_general(ea, we[...], (((2,), (1,)), ((), ())),
                                        preferred_element_type=jnp.float32)
                    + be_row[...][None])
                eref[...] = _bn_feat1(ea)

        for b in range(B):
            member = mem4[b]
            agg = (_dot(member, eknn4[b]) + _dot(lh[...], eloc4[b])) * dvr4[b]
            xb = x4[b]
            x4[b] = xb - ALPHA_V * (xb - agg)

    xa = x4[...]
    m = jnp.mean(xa, axis=(0, 1), keepdims=True)
    v = jnp.mean((xa - m) ** 2, axis=(0, 1), keepdims=True)
    xn = (xa - m) / jnp.sqrt(v + EPS)
    xb_ = xn * bnw_row[...][None] + bnb_row[...][None]
    out_ref[...] = jax.nn.relu(xb_) + x_in[...]


@functools.partial(jax.jit, static_argnames=())
def kernel(X, Wv, bv, We, be, bn_w, bn_b):
    lH = _local_incidence()
    lH_pad = np.zeros((N, E_LOC_PAD), np.float32)
    lH_pad[:, :E_LOC] = lH
    lHT_pad = np.ascontiguousarray(lH_pad.T)
    lDV = lH.sum(axis=1).reshape(N, 1).astype(np.float32)
    eye = np.eye(N, dtype=np.float32)

    f = pl.pallas_call(
        _hgnn_kernel,
        out_shape=jax.ShapeDtypeStruct((B, N, IN_CH), jnp.float32),
        scratch_shapes=[
            pltpu.VMEM((B, N, IN_CH), jnp.float32),
            pltpu.VMEM((N, N), jnp.int32),
            pltpu.VMEM((B, N, N), jnp.float32),
            pltpu.VMEM((B, N, IN_CH), jnp.float32),
            pltpu.VMEM((B, E_LOC_PAD, IN_CH), jnp.float32),
            pltpu.VMEM((B, N, 1), jnp.float32),
        ],
    )
    return f(X, Wv, bv.reshape(1, IN_CH), We, be.reshape(1, IN_CH),
             bn_w.reshape(1, IN_CH), bn_b.reshape(1, IN_CH),
             jnp.asarray(lH_pad), jnp.asarray(lHT_pad), jnp.asarray(lDV),
             jnp.asarray(eye))


# speedup vs baseline: 69.2289x; 1.2687x over previous
import jax
import jax.numpy as jnp
import numpy as np
from jax.experimental import pallas as pl
from jax.experimental.pallas import tpu as pltpu

IN_CH = 64
NODE = 32
K_NEIGS = 8
KS = 5
STRIDE = 2
N = NODE * NODE
B = 4
LAYER_NUM = 8
STEP = 4
ALPHA_V = 0.05
ALPHA_E = 0.9
EPS = 1e-5
E_LOC = 196
E_LOC_PAD = 256
F32_INF_BITS = 0x7F800000
I32_MAX = 0x7FFFFFFF


def _local_incidence():
    idx = np.arange(NODE * NODE).reshape(NODE, NODE)
    pats = []
    for i in range(0, NODE - KS + 1, STRIDE):
        for j in range(0, NODE - KS + 1, STRIDE):
            pats.append(idx[i:i + KS, j:j + KS].reshape(-1))
    inp_unf = np.stack(pats)
    e = inp_unf.shape[0]
    H = np.zeros((NODE * NODE, e), dtype=np.float32)
    for k in range(e):
        H[inp_unf[k], k] = 1.0
    return H


def _ninth_smallest(u_ref):
    iota_i = jax.lax.broadcasted_iota(jnp.int32, (N, N), 0)

    def step(_, carry):
        tv, ti = carry
        u = u_ref[...]
        after = (u > tv) | ((u == tv) & (iota_i > ti))
        mv = jnp.min(jnp.where(after, u, I32_MAX), axis=0, keepdims=True)
        mi = jnp.min(jnp.where(after & (u == mv), iota_i, N), axis=0,
                     keepdims=True)
        return mv, mi

    tv0 = jnp.full((1, N), -I32_MAX - 1, jnp.int32)
    ti0 = jnp.full((1, N), -1, jnp.int32)
    return jax.lax.fori_loop(0, K_NEIGS + 1, step, (tv0, ti0))


def _select_mask(u_ref, kvec):
    iota_i = jax.lax.broadcasted_iota(jnp.int32, (N, N), 0)

    def coarse(_, carry):
        lo, hi = carry
        mid = lo + (hi - lo) // 2
        cnt = jnp.sum(jnp.where(u_ref[...] <= mid, 1.0, 0.0), axis=0,
                      keepdims=True)
        ge = cnt >= kvec
        return jnp.where(ge, lo, mid + 1), jnp.where(ge, mid, hi)

    lo0 = jnp.zeros((1, N), jnp.int32)
    hi0 = jnp.full((1, N), F32_INF_BITS, jnp.int32)
    _, tstar = jax.lax.fori_loop(0, 31, coarse, (lo0, hi0))

    need = kvec - jnp.sum(jnp.where(u_ref[...] < tstar, 1.0, 0.0), axis=0,
                          keepdims=True)

    def fine(_, carry):
        lo, hi = carry
        mid = lo + (hi - lo) // 2
        cnt = jnp.sum(
            jnp.where((u_ref[...] == tstar) & (iota_i < mid), 1.0, 0.0),
            axis=0, keepdims=True)
        ge = cnt >= need
        return jnp.where(ge, lo, mid + 1), jnp.where(ge, mid, hi)

    lo0 = jnp.zeros((1, N), jnp.int32)
    hi0 = jnp.full((1, N), N, jnp.int32)
    _, istar = jax.lax.fori_loop(0, 10, fine, (lo0, hi0))

    u = u_ref[...]
    return (u < tstar) | ((u == tstar) & (iota_i < istar))


def _dotT(a, b):
    return jax.lax.dot_general(a, b, (((0,), (0,)), ((), ())),
                               preferred_element_type=jnp.float32)


def _dot(a, b):
    return jax.lax.dot_general(a, b, (((1,), (0,)), ((), ())),
                               preferred_element_type=jnp.float32)


def _bn_feat1(x):
    m = jnp.mean(x, axis=(0, 2), keepdims=True)
    v = jnp.mean((x - m) ** 2, axis=(0, 2), keepdims=True)
    return (x - m) / jnp.sqrt(v + EPS)


def _hgnn_kernel(x_in, wv, bv_row, we, be_row, bnw_row, bnb_row,
                 lh, lht, ldv_col, eye,
                 out_ref,
                 x4, u4, mem4, eknn4, eloc4, dvr4):
    diag = (jax.lax.broadcasted_iota(jnp.int32, (N, N), 0) ==
            jax.lax.broadcasted_iota(jnp.int32, (N, N), 1))
    diag_f = jnp.where(diag, 1.0, 0.0)
    iota_i = jax.lax.broadcasted_iota(jnp.int32, (N, N), 0)

    x4[...] = x_in[...]

    for layer in range(LAYER_NUM):
        for b in range(B):
            xb = x4[b]
            sq_col = jnp.sum(xb * xb, axis=1, keepdims=True)
            sq_row = _dotT(sq_col, eye[...])
            g = jax.lax.dot_general(xb, xb, (((1,), (1,)), ((), ())),
                                    preferred_element_type=jnp.float32)
            dt = (sq_row + (-2.0) * g) + sq_col
            uu = jax.lax.bitcast_convert_type(jnp.maximum(dt, 0.0), jnp.int32)
            u4[...] = jnp.maximum(uu, 0)

            tv9, ti9 = _ninth_smallest(u4)
            u = u4[...]
            mask9 = (u < tv9) | ((u == tv9) & (iota_i <= ti9))
            dv_col = jnp.sum(jnp.where(mask9, 1.0, 0.0), axis=1,
                             keepdims=True)
            kvec = _dotT(dv_col, eye[...])
            member = _select_mask(u4, kvec)
            memb = jnp.maximum(jnp.where(member, 1.0, 0.0), diag_f)
            mem4[b] = memb
            de_row = jnp.sum(memb, axis=0, keepdims=True)
            de_col = jax.lax.dot_general(
                eye[...], de_row, (((1,), (1,)), ((), ())),
                preferred_element_type=jnp.float32)
            eknn4[b] = _dotT(memb, xb) * (1.0 / de_col)
            eloc4[b] = _dot(lht[...], xb) * (1.0 / 25.0)
            dv_full = jnp.sum(memb, axis=1, keepdims=True) + ldv_col[...]
            dvr4[b] = 1.0 / dv_full

        if layer % STEP == 0:
            xa = x4[...]
            xa = xa + jax.nn.relu(
                jax.lax.dot_general(xa, wv[...], (((2,), (1,)), ((), ())),
                                    preferred_element_type=jnp.float32)
                + bv_row[...][None])
            x4[...] = _bn_feat1(xa)
            for eref in (eknn4, eloc4):
                ea = eref[...]
                ea = ea + jax.nn.relu(
                    jax.lax.dot_general(ea, we[...], (((2,), (1,)), ((), ())),
                                        preferred_element_type=jnp.float32)
                    + be_row[...][None])
                eref[...] = _bn_feat1(ea)

        for b in range(B):
            memb = mem4[b]
            agg = (_dot(memb, eknn4[b]) + _dot(lh[...], eloc4[b])) * dvr4[b]
            xb = x4[b]
            x4[b] = xb - ALPHA_V * (xb - agg)

    xa = x4[...]
    m = jnp.mean(xa, axis=(0, 1), keepdims=True)
    v = jnp.mean((xa - m) ** 2, axis=(0, 1), keepdims=True)
    xn = (xa - m) / jnp.sqrt(v + EPS)
    xb_ = xn * bnw_row[...][None] + bnb_row[...][None]
    out_ref[...] = jax.nn.relu(xb_) + x_in[...]


def kernel(X, Wv, bv, We, be, bn_w, bn_b):
    lH = _local_incidence()
    lH_pad = np.zeros((N, E_LOC_PAD), np.float32)
    lH_pad[:, :E_LOC] = lH
    lHT_pad = np.ascontiguousarray(lH_pad.T)
    lDV = lH.sum(axis=1).reshape(N, 1).astype(np.float32)
    eye = np.eye(N, dtype=np.float32)

    f = pl.pallas_call(
        _hgnn_kernel,
        out_shape=jax.ShapeDtypeStruct((B, N, IN_CH), jnp.float32),
        scratch_shapes=[
            pltpu.VMEM((B, N, IN_CH), jnp.float32),
            pltpu.VMEM((N, N), jnp.int32),
            pltpu.VMEM((B, N, N), jnp.float32),
            pltpu.VMEM((B, N, IN_CH), jnp.float32),
            pltpu.VMEM((B, E_LOC_PAD, IN_CH), jnp.float32),
            pltpu.VMEM((B, N, 1), jnp.float32),
        ],
    )
    return f(X, Wv, bv.reshape(1, IN_CH), We, be.reshape(1, IN_CH),
             bn_w.reshape(1, IN_CH), bn_b.reshape(1, IN_CH),
             jnp.asarray(lH_pad), jnp.asarray(lHT_pad), jnp.asarray(lDV),
             jnp.asarray(eye))


# speedup vs baseline: 70.6850x; 1.0210x over previous
import jax
import jax.numpy as jnp
import numpy as np
from jax.experimental import pallas as pl
from jax.experimental.pallas import tpu as pltpu

IN_CH = 64
NODE = 32
K_NEIGS = 8
KS = 5
STRIDE = 2
N = NODE * NODE
B = 4
LAYER_NUM = 8
STEP = 4
ALPHA_V = 0.05
ALPHA_E = 0.9
EPS = 1e-5
E_LOC = 196
E_LOC_PAD = 256
F32_INF_BITS = 0x7F800000
I32_MAX = 0x7FFFFFFF


def _local_incidence():
    idx = np.arange(NODE * NODE).reshape(NODE, NODE)
    pats = []
    for i in range(0, NODE - KS + 1, STRIDE):
        for j in range(0, NODE - KS + 1, STRIDE):
            pats.append(idx[i:i + KS, j:j + KS].reshape(-1))
    inp_unf = np.stack(pats)
    e = inp_unf.shape[0]
    H = np.zeros((NODE * NODE, e), dtype=np.float32)
    for k in range(e):
        H[inp_unf[k], k] = 1.0
    return H


def _ninth_smallest(u_ref):
    iota_i = jax.lax.broadcasted_iota(jnp.int32, (N, N), 0)

    def step(_, carry):
        tv, ti = carry
        u = u_ref[...]
        after = (u > tv) | ((u == tv) & (iota_i > ti))
        mv = jnp.min(jnp.where(after, u, I32_MAX), axis=0, keepdims=True)
        mi = jnp.min(jnp.where(after & (u == mv), iota_i, N), axis=0,
                     keepdims=True)
        return mv, mi

    tv0 = jnp.full((1, N), -I32_MAX - 1, jnp.int32)
    ti0 = jnp.full((1, N), -1, jnp.int32)
    return jax.lax.fori_loop(0, K_NEIGS + 1, step, (tv0, ti0))


def _select_mask(u_ref, kvec):
    iota_i = jax.lax.broadcasted_iota(jnp.int32, (N, N), 0)

    def coarse_cond(carry):
        lo, hi = carry
        return jnp.max(hi - lo) > 0

    def coarse(carry):
        lo, hi = carry
        mid = lo + (hi - lo) // 2
        cnt = jnp.sum(jnp.where(u_ref[...] <= mid, 1.0, 0.0), axis=0,
                      keepdims=True)
        ge = cnt >= kvec
        return jnp.where(ge, lo, mid + 1), jnp.where(ge, mid, hi)

    lo0 = jnp.min(u_ref[...], axis=0, keepdims=True)
    hi0 = jnp.max(u_ref[...], axis=0, keepdims=True)
    _, tstar = jax.lax.while_loop(coarse_cond, coarse, (lo0, hi0))

    need = kvec - jnp.sum(jnp.where(u_ref[...] < tstar, 1.0, 0.0), axis=0,
                          keepdims=True)
    eq_cnt = jnp.sum(jnp.where(u_ref[...] == tstar, 1.0, 0.0), axis=0,
                     keepdims=True)
    has_tie = jnp.max(eq_cnt - need) > 0.0

    def fine_path(_):
        def fine(i, carry):
            lo, hi = carry
            mid = lo + (hi - lo) // 2
            cnt = jnp.sum(
                jnp.where((u_ref[...] == tstar) & (iota_i < mid), 1.0, 0.0),
                axis=0, keepdims=True)
            ge = cnt >= need
            return jnp.where(ge, lo, mid + 1), jnp.where(ge, mid, hi)

        lo0 = jnp.zeros((1, N), jnp.int32)
        hi0 = jnp.full((1, N), N, jnp.int32)
        return jax.lax.fori_loop(0, 10, fine, (lo0, hi0))[1]

    istar = jax.lax.cond(has_tie, fine_path,
                         lambda _: jnp.full((1, N), N, jnp.int32), None)

    u = u_ref[...]
    return (u < tstar) | ((u == tstar) & (iota_i < istar))


def _dotT(a, b):
    return jax.lax.dot_general(a, b, (((0,), (0,)), ((), ())),
                               preferred_element_type=jnp.float32)


def _dot(a, b):
    return jax.lax.dot_general(a, b, (((1,), (0,)), ((), ())),
                               preferred_element_type=jnp.float32)


def _bn_feat1(x):
    m = jnp.mean(x, axis=(0, 2), keepdims=True)
    v = jnp.mean((x - m) ** 2, axis=(0, 2), keepdims=True)
    return (x - m) / jnp.sqrt(v + EPS)


def _hgnn_kernel(x_in, wv, bv_row, we, be_row, bnw_row, bnb_row,
                 lh, lht, ldv_col, eye,
                 out_ref,
                 x4, u4, mem4, eknn4, eloc4, dvr4):
    diag = (jax.lax.broadcasted_iota(jnp.int32, (N, N), 0) ==
            jax.lax.broadcasted_iota(jnp.int32, (N, N), 1))
    diag_f = jnp.where(diag, 1.0, 0.0)
    iota_i = jax.lax.broadcasted_iota(jnp.int32, (N, N), 0)

    x4[...] = x_in[...]

    for layer in range(LAYER_NUM):
        for b in range(B):
            xb = x4[b]
            sq_col = jnp.sum(xb * xb, axis=1, keepdims=True)
            sq_row = _dotT(sq_col, eye[...])
            g = jax.lax.dot_general(xb, xb, (((1,), (1,)), ((), ())),
                                    preferred_element_type=jnp.float32)
            dt = (sq_row + (-2.0) * g) + sq_col
            uu = jax.lax.bitcast_convert_type(jnp.maximum(dt, 0.0), jnp.int32)
            u4[...] = jnp.maximum(uu, 0)

            tv9, ti9 = _ninth_smallest(u4)
            u = u4[...]
            mask9 = (u < tv9) | ((u == tv9) & (iota_i <= ti9))
            dv_col = jnp.sum(jnp.where(mask9, 1.0, 0.0), axis=1,
                             keepdims=True)
            kvec = _dotT(dv_col, eye[...])
            member = _select_mask(u4, kvec)
            memb = jnp.maximum(jnp.where(member, 1.0, 0.0), diag_f)
            mem4[b] = memb
            de_row = jnp.sum(memb, axis=0, keepdims=True)
            de_col = jax.lax.dot_general(
                eye[...], de_row, (((1,), (1,)), ((), ())),
                preferred_element_type=jnp.float32)
            eknn4[b] = _dotT(memb, xb) * (1.0 / de_col)
            eloc4[b] = _dot(lht[...], xb) * (1.0 / 25.0)
            dv_full = jnp.sum(memb, axis=1, keepdims=True) + ldv_col[...]
            dvr4[b] = 1.0 / dv_full

        if layer % STEP == 0:
            xa = x4[...]
            xa = xa + jax.nn.relu(
                jax.lax.dot_general(xa, wv[...], (((2,), (1,)), ((), ())),
                                    preferred_element_type=jnp.float32)
                + bv_row[...][None])
            x4[...] = _bn_feat1(xa)
            for eref in (eknn4, eloc4):
                ea = eref[...]
                ea = ea + jax.nn.relu(
                    jax.lax.dot_general(ea, we[...], (((2,), (1,)), ((), ())),
                                        preferred_element_type=jnp.float32)
                    + be_row[...][None])
                eref[...] = _bn_feat1(ea)

        for b in range(B):
            memb = mem4[b]
            agg = (_dot(memb, eknn4[b]) + _dot(lh[...], eloc4[b])) * dvr4[b]
            xb = x4[b]
            x4[b] = xb - ALPHA_V * (xb - agg)

    xa = x4[...]
    m = jnp.mean(xa, axis=(0, 1), keepdims=True)
    v = jnp.mean((xa - m) ** 2, axis=(0, 1), keepdims=True)
    xn = (xa - m) / jnp.sqrt(v + EPS)
    xb_ = xn * bnw_row[...][None] + bnb_row[...][None]
    out_ref[...] = jax.nn.relu(xb_) + x_in[...]


def kernel(X, Wv, bv, We, be, bn_w, bn_b):
    lH = _local_incidence()
    lH_pad = np.zeros((N, E_LOC_PAD), np.float32)
    lH_pad[:, :E_LOC] = lH
    lHT_pad = np.ascontiguousarray(lH_pad.T)
    lDV = lH.sum(axis=1).reshape(N, 1).astype(np.float32)
    eye = np.eye(N, dtype=np.float32)

    f = pl.pallas_call(
        _hgnn_kernel,
        out_shape=jax.ShapeDtypeStruct((B, N, IN_CH), jnp.float32),
        scratch_shapes=[
            pltpu.VMEM((B, N, IN_CH), jnp.float32),
            pltpu.VMEM((N, N), jnp.int32),
            pltpu.VMEM((B, N, N), jnp.float32),
            pltpu.VMEM((B, N, IN_CH), jnp.float32),
            pltpu.VMEM((B, E_LOC_PAD, IN_CH), jnp.float32),
            pltpu.VMEM((B, N, 1), jnp.float32),
        ],
    )
    return f(X, Wv, bv.reshape(1, IN_CH), We, be.reshape(1, IN_CH),
             bn_w.reshape(1, IN_CH), bn_b.reshape(1, IN_CH),
             jnp.asarray(lH_pad), jnp.asarray(lHT_pad), jnp.asarray(lDV),
             jnp.asarray(eye))


# speedup vs baseline: 107.8961x; 1.5264x over previous
import jax
import jax.numpy as jnp
import numpy as np
from jax.experimental import pallas as pl
from jax.experimental.pallas import tpu as pltpu

IN_CH = 64
NODE = 32
K_NEIGS = 8
KS = 5
STRIDE = 2
N = NODE * NODE
B = 4
LAYER_NUM = 8
STEP = 4
ALPHA_V = 0.05
ALPHA_E = 0.9
EPS = 1e-5
E_LOC = 196
E_LOC_PAD = 256
F32_INF_BITS = 0x7F800000
I32_MAX = 0x7FFFFFFF


def _local_incidence():
    idx = np.arange(NODE * NODE).reshape(NODE, NODE)
    pats = []
    for i in range(0, NODE - KS + 1, STRIDE):
        for j in range(0, NODE - KS + 1, STRIDE):
            pats.append(idx[i:i + KS, j:j + KS].reshape(-1))
    inp_unf = np.stack(pats)
    e = inp_unf.shape[0]
    H = np.zeros((NODE * NODE, e), dtype=np.float32)
    for k in range(e):
        H[inp_unf[k], k] = 1.0
    return H


def _ninth_smallest(u_ref):
    iota_i = jax.lax.broadcasted_iota(jnp.int32, (N, N), 0)

    def step(_, carry):
        tv, ti = carry
        u = u_ref[...]
        after = (u > tv) | ((u == tv) & (iota_i > ti))
        mv = jnp.min(jnp.where(after, u, I32_MAX), axis=0, keepdims=True)
        mi = jnp.min(jnp.where(after & (u == mv), iota_i, N), axis=0,
                     keepdims=True)
        return mv, mi

    tv0 = jnp.full((1, N), -I32_MAX - 1, jnp.int32)
    ti0 = jnp.full((1, N), -1, jnp.int32)
    return jax.lax.fori_loop(0, K_NEIGS + 1, step, (tv0, ti0))


def _knn_degree(u_ref):
    def vstep(_, tv):
        u = u_ref[...]
        return jnp.min(jnp.where(u > tv, u, I32_MAX), axis=0, keepdims=True)

    tv0 = jnp.full((1, N), -I32_MAX - 1, jnp.int32)
    tv9 = jax.lax.fori_loop(0, K_NEIGS + 1, vstep, tv0)
    cnt9 = jnp.sum(jnp.where(u_ref[...] <= tv9, 1.0, 0.0), axis=0,
                   keepdims=True)
    clean = jnp.max(jnp.abs(cnt9 - 9.0)) == 0.0

    def fast(_):
        m9 = u_ref[...] <= tv9
        return jnp.sum(jnp.where(m9, 1.0, 0.0), axis=1, keepdims=True)

    def slow(_):
        tv, ti = _ninth_smallest(u_ref)
        iota_i = jax.lax.broadcasted_iota(jnp.int32, (N, N), 0)
        u = u_ref[...]
        m9 = (u < tv) | ((u == tv) & (iota_i <= ti))
        return jnp.sum(jnp.where(m9, 1.0, 0.0), axis=1, keepdims=True)

    return jax.lax.cond(clean, fast, slow, None)


def _select_mask(u_ref, kvec):
    iota_i = jax.lax.broadcasted_iota(jnp.int32, (N, N), 0)

    def coarse_cond(carry):
        lo, hi = carry
        return jnp.max(hi - lo) > 0

    def coarse(carry):
        lo, hi = carry
        mid = lo + (hi - lo) // 2
        cnt = jnp.sum(jnp.where(u_ref[...] <= mid, 1.0, 0.0), axis=0,
                      keepdims=True)
        ge = cnt >= kvec
        return jnp.where(ge, lo, mid + 1), jnp.where(ge, mid, hi)

    lo0 = jnp.min(u_ref[...], axis=0, keepdims=True)
    hi0 = jnp.max(u_ref[...], axis=0, keepdims=True)
    _, tstar = jax.lax.while_loop(coarse_cond, coarse, (lo0, hi0))

    need = kvec - jnp.sum(jnp.where(u_ref[...] < tstar, 1.0, 0.0), axis=0,
                          keepdims=True)
    eq_cnt = jnp.sum(jnp.where(u_ref[...] == tstar, 1.0, 0.0), axis=0,
                     keepdims=True)
    has_tie = jnp.max(eq_cnt - need) > 0.0

    def fine_path(_):
        def fine(i, carry):
            lo, hi = carry
            mid = lo + (hi - lo) // 2
            cnt = jnp.sum(
                jnp.where((u_ref[...] == tstar) & (iota_i < mid), 1.0, 0.0),
                axis=0, keepdims=True)
            ge = cnt >= need
            return jnp.where(ge, lo, mid + 1), jnp.where(ge, mid, hi)

        lo0 = jnp.zeros((1, N), jnp.int32)
        hi0 = jnp.full((1, N), N, jnp.int32)
        return jax.lax.fori_loop(0, 10, fine, (lo0, hi0))[1]

    istar = jax.lax.cond(has_tie, fine_path,
                         lambda _: jnp.full((1, N), N, jnp.int32), None)

    u = u_ref[...]
    return (u < tstar) | ((u == tstar) & (iota_i < istar))


def _dotT(a, b):
    return jax.lax.dot_general(a, b, (((0,), (0,)), ((), ())),
                               preferred_element_type=jnp.float32)


def _dot(a, b):
    return jax.lax.dot_general(a, b, (((1,), (0,)), ((), ())),
                               preferred_element_type=jnp.float32)


def _bn_feat1(x):
    m = jnp.mean(x, axis=(0, 2), keepdims=True)
    v = jnp.mean((x - m) ** 2, axis=(0, 2), keepdims=True)
    return (x - m) / jnp.sqrt(v + EPS)


def _hgnn_kernel(x_in, wv, bv_row, we, be_row, bnw_row, bnb_row,
                 lh, lht, ldv_col, eye,
                 out_ref,
                 x4, u4, mem4, eknn4, eloc4, dvr4):
    diag = (jax.lax.broadcasted_iota(jnp.int32, (N, N), 0) ==
            jax.lax.broadcasted_iota(jnp.int32, (N, N), 1))
    diag_f = jnp.where(diag, 1.0, 0.0)

    x4[...] = x_in[...]

    for layer in range(LAYER_NUM):
        for b in range(B):
            xb = x4[b]
            sq_col = jnp.sum(xb * xb, axis=1, keepdims=True)
            sq_row = _dotT(sq_col, eye[...])
            g = jax.lax.dot_general(xb, xb, (((1,), (1,)), ((), ())),
                                    preferred_element_type=jnp.float32)
            dt = (sq_row + (-2.0) * g) + sq_col
            uu = jax.lax.bitcast_convert_type(jnp.maximum(dt, 0.0), jnp.int32)
            u4[...] = jnp.maximum(uu, 0)

            dv_col = _knn_degree(u4)
            kvec = _dotT(dv_col, eye[...])
            member = _select_mask(u4, kvec)
            memb = jnp.maximum(jnp.where(member, 1.0, 0.0), diag_f)
            mem4[b] = memb
            de_row = jnp.sum(memb, axis=0, keepdims=True)
            de_col = jax.lax.dot_general(
                eye[...], de_row, (((1,), (1,)), ((), ())),
                preferred_element_type=jnp.float32)
            eknn4[b] = _dotT(memb, xb) * (1.0 / de_col)
            eloc4[b] = _dot(lht[...], xb) * (1.0 / 25.0)
            dv_full = jnp.sum(memb, axis=1, keepdims=True) + ldv_col[...]
            dvr4[b] = 1.0 / dv_full

        if layer % STEP == 0:
            xa = x4[...]
            xa = xa + jax.nn.relu(
                jax.lax.dot_general(xa, wv[...], (((2,), (1,)), ((), ())),
                                    preferred_element_type=jnp.float32)
                + bv_row[...][None])
            x4[...] = _bn_feat1(xa)
            for eref in (eknn4, eloc4):
                ea = eref[...]
                ea = ea + jax.nn.relu(
                    jax.lax.dot_general(ea, we[...], (((2,), (1,)), ((), ())),
                                        preferred_element_type=jnp.float32)
                    + be_row[...][None])
                eref[...] = _bn_feat1(ea)

        for b in range(B):
            memb = mem4[b]
            agg = (_dot(memb, eknn4[b]) + _dot(lh[...], eloc4[b])) * dvr4[b]
            xb = x4[b]
            x4[b] = xb - ALPHA_V * (xb - agg)

    xa = x4[...]
    m = jnp.mean(xa, axis=(0, 1), keepdims=True)
    v = jnp.mean((xa - m) ** 2, axis=(0, 1), keepdims=True)
    xn = (xa - m) / jnp.sqrt(v + EPS)
    xb_ = xn * bnw_row[...][None] + bnb_row[...][None]
    out_ref[...] = jax.nn.relu(xb_) + x_in[...]


def kernel(X, Wv, bv, We, be, bn_w, bn_b):
    lH = _local_incidence()
    lH_pad = np.zeros((N, E_LOC_PAD), np.float32)
    lH_pad[:, :E_LOC] = lH
    lHT_pad = np.ascontiguousarray(lH_pad.T)
    lDV = lH.sum(axis=1).reshape(N, 1).astype(np.float32)
    eye = np.eye(N, dtype=np.float32)

    f = pl.pallas_call(
        _hgnn_kernel,
        out_shape=jax.ShapeDtypeStruct((B, N, IN_CH), jnp.float32),
        scratch_shapes=[
            pltpu.VMEM((B, N, IN_CH), jnp.float32),
            pltpu.VMEM((N, N), jnp.int32),
            pltpu.VMEM((B, N, N), jnp.float32),
            pltpu.VMEM((B, N, IN_CH), jnp.float32),
            pltpu.VMEM((B, E_LOC_PAD, IN_CH), jnp.float32),
            pltpu.VMEM((B, N, 1), jnp.float32),
        ],
    )
    return f(X, Wv, bv.reshape(1, IN_CH), We, be.reshape(1, IN_CH),
             bn_w.reshape(1, IN_CH), bn_b.reshape(1, IN_CH),
             jnp.asarray(lH_pad), jnp.asarray(lHT_pad), jnp.asarray(lDV),
             jnp.asarray(eye))
